# Initial kernel scaffold; baseline (speedup 1.0000x reference)
import jax
import jax.numpy as jnp
from jax.experimental import pallas as pl


def kernel(x, mem_kv_db, Wq, Wkv, Wout, scale):
    raise NotImplementedError("write your pallas kernel here")


# speedup vs baseline: 12.7590x; 12.7590x over previous
import functools
import math

import jax
import jax.numpy as jnp
from jax.experimental import pallas as pl
from jax.experimental.pallas import tpu as pltpu

DH = 64
K = 32


_HI = jax.lax.Precision.HIGHEST


def _doth(a, b):
    return jnp.dot(a, b, preferred_element_type=jnp.float32, precision=_HI)


def _dotd(a, b):
    return jnp.dot(a, b, preferred_element_type=jnp.float32)


def _proj_kernel(x_ref, wq_ref, wkv_ref, hseg_ref, hsegT_ref, qn_ref, kn_ref, v_ref):
    x = x_ref[...]
    q = _dotd(x, wq_ref[...])
    kv = _dotd(x, wkv_ref[...])
    k = kv[:, :DH]
    v = kv[:, DH:]
    s2 = _doth(q * q, hseg_ref[...])
    norm = jnp.sqrt(s2)
    normfull = _doth(norm, hsegT_ref[...])
    qn_ref[...] = q / jnp.maximum(normfull, 1e-12)
    kn = k / jnp.maximum(jnp.sqrt(jnp.sum(k * k, axis=1, keepdims=True)), 1e-12)
    kn_ref[...] = kn
    v_ref[...] = v


def _attn_kernel(scale_ref, qn_ref, knT_ref, v_ref, mkT_ref, mv_ref, wout_ref,
                 o_ref, *, bq, n):
    h = pl.program_id(1)
    qb = pl.program_id(0)
    sc = jnp.exp(scale_ref[0])

    q = qn_ref[0]
    s = _doth(q, mkT_ref[...])

    b = jax.lax.bitcast_convert_type(s, jnp.int32)
    u = b ^ ((b >> 31) & jnp.int32(0x7FFFFFFF))

    def count_ge(t):
        return jnp.sum((u >= t).astype(jnp.int32), axis=1, keepdims=True)

    p0 = jnp.where(count_ge(jnp.zeros((q.shape[0], 1), jnp.int32)) >= K,
                   jnp.int32(0), jnp.int32(-2147483648))

    def body(i, p):
        cand = p | (jnp.int32(1) << (jnp.int32(30) - i))
        return jnp.where(count_ge(cand) >= K, cand, p)

    p = jax.lax.fori_loop(0, 31, body, p0)
    mask = u >= p

    m_raw = jnp.max(jnp.where(mask, s, -jnp.inf), axis=1, keepdims=True)
    w = jnp.where(mask, jnp.exp(sc * (s - m_raw)), 0.0)
    zmem = jnp.sum(w, axis=1, keepdims=True)

    logits = _dotd(q, knT_ref[...]) * sc
    rows = qb * bq + jax.lax.broadcasted_iota(jnp.int32, logits.shape, 0)
    cols = jax.lax.broadcasted_iota(jnp.int32, logits.shape, 1)
    logits = jnp.where(cols > rows, -jnp.inf, logits)
    mloc = jnp.max(logits, axis=1, keepdims=True)
    pvals = jnp.exp(logits - mloc)
    zloc = jnp.sum(pvals, axis=1, keepdims=True)

    gmem = sc * m_raw
    g = jnp.maximum(gmem, mloc)
    a = jnp.exp(gmem - g)
    bt = jnp.exp(mloc - g)
    z = zmem * a + zloc * bt
    accmem = _doth(w * (a / z), mv_ref[...])
    accloc = _dotd(pvals * (bt / z), v_ref[...])
    o = accmem + accloc

    contrib = _dotd(o, wout_ref[...])

    @pl.when(h == 0)
    def _():
        o_ref[...] = jnp.zeros_like(o_ref)

    o_ref[...] += contrib


def kernel(x, mem_kv_db, Wq, Wkv, Wout, scale):
    bsz, n, d = x.shape
    m = mem_kv_db.shape[1]
    h = scale.shape[0]
    x2 = x.reshape(n, d)

    hseg = jnp.repeat(jnp.eye(h, dtype=jnp.float32), DH, axis=0)
    hsegT = hseg.T

    br = 512 if n % 512 == 0 else n
    qn, kn, v = pl.pallas_call(
        _proj_kernel,
        grid=(n // br,),
        in_specs=[
            pl.BlockSpec((br, d), lambda r: (r, 0)),
            pl.BlockSpec((d, h * DH), lambda r: (0, 0)),
            pl.BlockSpec((d, 2 * DH), lambda r: (0, 0)),
            pl.BlockSpec((d, h), lambda r: (0, 0)),
            pl.BlockSpec((h, d), lambda r: (0, 0)),
        ],
        out_specs=[
            pl.BlockSpec((br, h * DH), lambda r: (r, 0)),
            pl.BlockSpec((br, DH), lambda r: (r, 0)),
            pl.BlockSpec((br, DH), lambda r: (r, 0)),
        ],
        out_shape=[
            jax.ShapeDtypeStruct((n, h * DH), jnp.float32),
            jax.ShapeDtypeStruct((n, DH), jnp.float32),
            jax.ShapeDtypeStruct((n, DH), jnp.float32),
        ],
    )(x2, Wq, Wkv, hseg, hsegT)

    qn3 = qn.reshape(n, h, DH).transpose(1, 0, 2)
    knT = kn.T
    mkT = mem_kv_db[0, :, 0, :].T
    mv = mem_kv_db[0, :, 1, :]
    sc2 = scale.reshape(h, 1, 1).astype(jnp.float32)

    bq = 256 if n % 256 == 0 else n
    nq = n // bq
    out = pl.pallas_call(
        functools.partial(_attn_kernel, bq=bq, n=n),
        grid=(nq, h),
        in_specs=[
            pl.BlockSpec((1, 1, 1), lambda qb, hh: (hh, 0, 0)),
            pl.BlockSpec((1, bq, DH), lambda qb, hh: (hh, qb, 0)),
            pl.BlockSpec((DH, n), lambda qb, hh: (0, 0)),
            pl.BlockSpec((n, DH), lambda qb, hh: (0, 0)),
            pl.BlockSpec((DH, m), lambda qb, hh: (0, 0)),
            pl.BlockSpec((m, DH), lambda qb, hh: (0, 0)),
            pl.BlockSpec((DH, d), lambda qb, hh: (hh, 0)),
        ],
        out_specs=pl.BlockSpec((bq, d), lambda qb, hh: (qb, 0)),
        out_shape=jax.ShapeDtypeStruct((n, d), jnp.float32),
        compiler_params=pltpu.CompilerParams(
            dimension_semantics=("parallel", "arbitrary"),
        ),
    )(sc2, qn3, knT, v, mkT, mv, Wout)

    return out.reshape(bsz, n, d)


# speedup vs baseline: 14.3127x; 1.1218x over previous
import functools
import math

import jax
import jax.numpy as jnp
from jax.experimental import pallas as pl
from jax.experimental.pallas import tpu as pltpu

DH = 64
K = 32


_HI = jax.lax.Precision.HIGHEST


def _doth(a, b):
    return jnp.dot(a, b, preferred_element_type=jnp.float32, precision=_HI)


def _dotd(a, b):
    return jnp.dot(a, b, preferred_element_type=jnp.float32)


def _proj_kernel(x_ref, wq_ref, wkv_ref, hseg_ref, hsegT_ref, qn_ref, kn_ref, v_ref):
    x = x_ref[...]
    q = _dotd(x, wq_ref[...])
    kv = _dotd(x, wkv_ref[...])
    k = kv[:, :DH]
    v = kv[:, DH:]
    s2 = _doth(q * q, hseg_ref[...])
    norm = jnp.sqrt(s2)
    normfull = _doth(norm, hsegT_ref[...])
    qn_ref[...] = q / jnp.maximum(normfull, 1e-12)
    kn = k / jnp.maximum(jnp.sqrt(jnp.sum(k * k, axis=1, keepdims=True)), 1e-12)
    kn_ref[...] = kn
    v_ref[...] = v


def _attn_kernel(scale_ref, qn_ref, knT_ref, v_ref, mkT_ref, mv_ref, wout_ref,
                 o_ref, *, bq, n):
    h = pl.program_id(1)
    qb = pl.program_id(0)
    sc = jnp.exp(scale_ref[0])

    q = qn_ref[0]
    s = _doth(q, mkT_ref[...])

    b = jax.lax.bitcast_convert_type(s, jnp.int32)
    u = b ^ ((b >> 31) & jnp.int32(0x7FFFFFFF))

    def count_ge(t):
        return jnp.sum((u >= t).astype(jnp.int32), axis=1, keepdims=True)

    c0 = count_ge(jnp.zeros((q.shape[0], 1), jnp.int32))
    has = c0 >= K
    p0 = jnp.where(has, jnp.int32(0), jnp.int32(-2147483648))
    n0 = jnp.where(has, c0, jnp.full_like(c0, u.shape[1]))

    def cond(st):
        i, _, c = st
        return jnp.logical_and(i < 31, jnp.any(c != K))

    def body(st):
        i, p, c = st
        cand = p | (jnp.int32(1) << (jnp.int32(30) - i))
        cnt = count_ge(cand)
        take = cnt >= K
        return (i + 1, jnp.where(take, cand, p), jnp.where(take, cnt, c))

    _, p, _ = jax.lax.while_loop(cond, body, (jnp.int32(0), p0, n0))
    mask = u >= p

    m_raw = jnp.max(jnp.where(mask, s, -jnp.inf), axis=1, keepdims=True)
    w = jnp.where(mask, jnp.exp(sc * (s - m_raw)), 0.0)
    zmem = jnp.sum(w, axis=1, keepdims=True)

    logits = _dotd(q, knT_ref[...]) * sc
    rows = qb * bq + jax.lax.broadcasted_iota(jnp.int32, logits.shape, 0)
    cols = jax.lax.broadcasted_iota(jnp.int32, logits.shape, 1)
    logits = jnp.where(cols > rows, -jnp.inf, logits)
    mloc = jnp.max(logits, axis=1, keepdims=True)
    pvals = jnp.exp(logits - mloc)
    zloc = jnp.sum(pvals, axis=1, keepdims=True)

    gmem = sc * m_raw
    g = jnp.maximum(gmem, mloc)
    a = jnp.exp(gmem - g)
    bt = jnp.exp(mloc - g)
    z = zmem * a + zloc * bt
    accmem = _doth(w * (a / z), mv_ref[...])
    accloc = _dotd(pvals * (bt / z), v_ref[...])
    o = accmem + accloc

    contrib = _dotd(o, wout_ref[...])

    @pl.when(h == 0)
    def _():
        o_ref[...] = jnp.zeros_like(o_ref)

    o_ref[...] += contrib


def kernel(x, mem_kv_db, Wq, Wkv, Wout, scale):
    bsz, n, d = x.shape
    m = mem_kv_db.shape[1]
    h = scale.shape[0]
    x2 = x.reshape(n, d)

    hseg = jnp.repeat(jnp.eye(h, dtype=jnp.float32), DH, axis=0)
    hsegT = hseg.T

    br = 512 if n % 512 == 0 else n
    qn, kn, v = pl.pallas_call(
        _proj_kernel,
        grid=(n // br,),
        in_specs=[
            pl.BlockSpec((br, d), lambda r: (r, 0)),
            pl.BlockSpec((d, h * DH), lambda r: (0, 0)),
            pl.BlockSpec((d, 2 * DH), lambda r: (0, 0)),
            pl.BlockSpec((d, h), lambda r: (0, 0)),
            pl.BlockSpec((h, d), lambda r: (0, 0)),
        ],
        out_specs=[
            pl.BlockSpec((br, h * DH), lambda r: (r, 0)),
            pl.BlockSpec((br, DH), lambda r: (r, 0)),
            pl.BlockSpec((br, DH), lambda r: (r, 0)),
        ],
        out_shape=[
            jax.ShapeDtypeStruct((n, h * DH), jnp.float32),
            jax.ShapeDtypeStruct((n, DH), jnp.float32),
            jax.ShapeDtypeStruct((n, DH), jnp.float32),
        ],
    )(x2, Wq, Wkv, hseg, hsegT)

    qn3 = qn.reshape(n, h, DH).transpose(1, 0, 2)
    knT = kn.T
    mkT = mem_kv_db[0, :, 0, :].T
    mv = mem_kv_db[0, :, 1, :]
    sc2 = scale.reshape(h, 1, 1).astype(jnp.float32)

    bq = 512 if n % 512 == 0 else n
    nq = n // bq
    out = pl.pallas_call(
        functools.partial(_attn_kernel, bq=bq, n=n),
        grid=(nq, h),
        in_specs=[
            pl.BlockSpec((1, 1, 1), lambda qb, hh: (hh, 0, 0)),
            pl.BlockSpec((1, bq, DH), lambda qb, hh: (hh, qb, 0)),
            pl.BlockSpec((DH, n), lambda qb, hh: (0, 0)),
            pl.BlockSpec((n, DH), lambda qb, hh: (0, 0)),
            pl.BlockSpec((DH, m), lambda qb, hh: (0, 0)),
            pl.BlockSpec((m, DH), lambda qb, hh: (0, 0)),
            pl.BlockSpec((DH, d), lambda qb, hh: (hh, 0)),
        ],
        out_specs=pl.BlockSpec((bq, d), lambda qb, hh: (qb, 0)),
        out_shape=jax.ShapeDtypeStruct((n, d), jnp.float32),
        compiler_params=pltpu.CompilerParams(
            dimension_semantics=("parallel", "arbitrary"),
        ),
    )(sc2, qn3, knT, v, mkT, mv, Wout)

    return out.reshape(bsz, n, d)


# speedup vs baseline: 19.0262x; 1.3293x over previous
import functools
import math

import jax
import jax.numpy as jnp
from jax.experimental import pallas as pl
from jax.experimental.pallas import tpu as pltpu

DH = 64
K = 32


_HI = jax.lax.Precision.HIGHEST


def _doth(a, b):
    ah = a.astype(jnp.bfloat16)
    al = (a - ah.astype(jnp.float32)).astype(jnp.bfloat16)
    bh = b.astype(jnp.bfloat16)
    bl = (b - bh.astype(jnp.float32)).astype(jnp.bfloat16)
    d = jnp.dot(ah, bh, preferred_element_type=jnp.float32)
    d = d + jnp.dot(ah, bl, preferred_element_type=jnp.float32)
    d = d + jnp.dot(al, bh, preferred_element_type=jnp.float32)
    return d


def _dotd(a, b):
    return jnp.dot(a, b, preferred_element_type=jnp.float32)


def _proj_kernel(x_ref, wq_ref, wkv_ref, hseg_ref, hsegT_ref, qn_ref, kn_ref, v_ref):
    x = x_ref[...]
    q = _dotd(x, wq_ref[...])
    kv = _dotd(x, wkv_ref[...])
    k = kv[:, :DH]
    v = kv[:, DH:]
    s2 = _doth(q * q, hseg_ref[...])
    norm = jnp.sqrt(s2)
    normfull = _doth(norm, hsegT_ref[...])
    qn_ref[...] = q / jnp.maximum(normfull, 1e-12)
    kn = k / jnp.maximum(jnp.sqrt(jnp.sum(k * k, axis=1, keepdims=True)), 1e-12)
    kn_ref[...] = kn
    v_ref[...] = v


def _attn_kernel(scale_ref, qn_ref, knT_ref, v_ref, mkT_ref, mv_ref, wout_ref,
                 o_ref, *, bq, n):
    h = pl.program_id(1)
    qb = pl.program_id(0)
    sc = jnp.exp(scale_ref[0])

    q = qn_ref[0]
    s = _doth(q, mkT_ref[...])

    b = jax.lax.bitcast_convert_type(s, jnp.int32)
    u = b ^ ((b >> 31) & jnp.int32(0x7FFFFFFF))

    def count_ge(t):
        return jnp.sum((u >= t).astype(jnp.int32), axis=1, keepdims=True)

    c0 = count_ge(jnp.zeros((q.shape[0], 1), jnp.int32))
    has = c0 >= K
    p0 = jnp.where(has, jnp.int32(0), jnp.int32(-2147483648))
    n0 = jnp.where(has, c0, jnp.full_like(c0, u.shape[1]))

    def cond(st):
        i, _, c = st
        return jnp.logical_and(i < 31, jnp.any(c != K))

    def body(st):
        i, p, c = st
        cand = p | (jnp.int32(1) << (jnp.int32(30) - i))
        cnt = count_ge(cand)
        take = cnt >= K
        return (i + 1, jnp.where(take, cand, p), jnp.where(take, cnt, c))

    _, p, _ = jax.lax.while_loop(cond, body, (jnp.int32(0), p0, n0))
    mask = u >= p

    m_raw = jnp.max(jnp.where(mask, s, -jnp.inf), axis=1, keepdims=True)
    w = jnp.where(mask, jnp.exp(sc * (s - m_raw)), 0.0)
    zmem = jnp.sum(w, axis=1, keepdims=True)

    logits = _dotd(q, knT_ref[...]) * sc
    rows = qb * bq + jax.lax.broadcasted_iota(jnp.int32, logits.shape, 0)
    cols = jax.lax.broadcasted_iota(jnp.int32, logits.shape, 1)
    logits = jnp.where(cols > rows, -jnp.inf, logits)
    mloc = jnp.max(logits, axis=1, keepdims=True)
    pvals = jnp.exp(logits - mloc)
    zloc = jnp.sum(pvals, axis=1, keepdims=True)

    gmem = sc * m_raw
    g = jnp.maximum(gmem, mloc)
    a = jnp.exp(gmem - g)
    bt = jnp.exp(mloc - g)
    z = zmem * a + zloc * bt
    accmem = _doth(w * (a / z), mv_ref[...])
    accloc = _dotd(pvals * (bt / z), v_ref[...])
    o = accmem + accloc

    contrib = _dotd(o, wout_ref[...])

    @pl.when(h == 0)
    def _():
        o_ref[...] = jnp.zeros_like(o_ref)

    o_ref[...] += contrib


def kernel(x, mem_kv_db, Wq, Wkv, Wout, scale):
    bsz, n, d = x.shape
    m = mem_kv_db.shape[1]
    h = scale.shape[0]
    x2 = x.reshape(n, d)

    hseg = jnp.repeat(jnp.eye(h, dtype=jnp.float32), DH, axis=0)
    hsegT = hseg.T

    br = 512 if n % 512 == 0 else n
    qn, kn, v = pl.pallas_call(
        _proj_kernel,
        grid=(n // br,),
        in_specs=[
            pl.BlockSpec((br, d), lambda r: (r, 0)),
            pl.BlockSpec((d, h * DH), lambda r: (0, 0)),
            pl.BlockSpec((d, 2 * DH), lambda r: (0, 0)),
            pl.BlockSpec((d, h), lambda r: (0, 0)),
            pl.BlockSpec((h, d), lambda r: (0, 0)),
        ],
        out_specs=[
            pl.BlockSpec((br, h * DH), lambda r: (r, 0)),
            pl.BlockSpec((br, DH), lambda r: (r, 0)),
            pl.BlockSpec((br, DH), lambda r: (r, 0)),
        ],
        out_shape=[
            jax.ShapeDtypeStruct((n, h * DH), jnp.float32),
            jax.ShapeDtypeStruct((n, DH), jnp.float32),
            jax.ShapeDtypeStruct((n, DH), jnp.float32),
        ],
    )(x2, Wq, Wkv, hseg, hsegT)

    qn3 = qn.reshape(n, h, DH).transpose(1, 0, 2)
    knT = kn.T
    mkT = mem_kv_db[0, :, 0, :].T
    mv = mem_kv_db[0, :, 1, :]
    sc2 = scale.reshape(h, 1, 1).astype(jnp.float32)

    bq = 512 if n % 512 == 0 else n
    nq = n // bq
    out = pl.pallas_call(
        functools.partial(_attn_kernel, bq=bq, n=n),
        grid=(nq, h),
        in_specs=[
            pl.BlockSpec((1, 1, 1), lambda qb, hh: (hh, 0, 0)),
            pl.BlockSpec((1, bq, DH), lambda qb, hh: (hh, qb, 0)),
            pl.BlockSpec((DH, n), lambda qb, hh: (0, 0)),
            pl.BlockSpec((n, DH), lambda qb, hh: (0, 0)),
            pl.BlockSpec((DH, m), lambda qb, hh: (0, 0)),
            pl.BlockSpec((m, DH), lambda qb, hh: (0, 0)),
            pl.BlockSpec((DH, d), lambda qb, hh: (hh, 0)),
        ],
        out_specs=pl.BlockSpec((bq, d), lambda qb, hh: (qb, 0)),
        out_shape=jax.ShapeDtypeStruct((n, d), jnp.float32),
        compiler_params=pltpu.CompilerParams(
            dimension_semantics=("parallel", "arbitrary"),
        ),
    )(sc2, qn3, knT, v, mkT, mv, Wout)

    return out.reshape(bsz, n, d)
